# baseline (speedup 1.0000x reference)
import functools

import jax
import jax.numpy as jnp
import numpy as np
from jax import lax
from jax.experimental import pallas as pl
from jax.experimental.pallas import tpu as pltpu
from jax.experimental.pallas import tpu_sc as plsc

H = 128
W = 2048
TIME_WARP_PARA = 40
FREQ_MASK_PARA = 27
TIME_MASK_PARA = 70
FREQ_MASK_NUM = 2
TIME_MASK_NUM = 2

NUM_WORKERS = 32
LANES = 16

ROWS_PER_WORKER = 8
COLS_PER_WORKER = W // 2
_CHUNK = ROWS_PER_WORKER * COLS_PER_WORKER
HALO_W = COLS_PER_WORKER + 128

_F = FREQ_MASK_PARA // 2
_T = TIME_MASK_PARA // 2
_ROW_MASKS = [((i + 1) * H // 4, (i + 1) * H // 4 + _F) for i in range(FREQ_MASK_NUM)]
_COL_MASKS = [((i + 1) * W // 4, (i + 1) * W // 4 + _T) for i in range(TIME_MASK_NUM)]


def _build_qtab():
    eps = 1e-10

    def phi(r):
        r = jnp.maximum(r, eps)
        return 0.5 * r * jnp.log(r)

    def cross_sq_dist(a, b):
        an = jnp.sum(a * a, axis=-1)[:, :, None]
        bn = jnp.sum(b * b, axis=-1)[:, None, :]
        ab = jnp.einsum('bnd,bmd->bnm', a, b)
        return an - 2.0 * ab + bn

    y = float(H // 2)
    pt = float(W // 2)
    dist = float(TIME_WARP_PARA // 2)
    src = jnp.array(
        [[[y, pt], [0.0, 0.0], [0.0, W - 1.0], [H - 1.0, 0.0], [H - 1.0, W - 1.0]]],
        dtype=jnp.float32)
    dst = src.at[0, 0, 1].add(dist)
    flows = dst - src

    c = dst
    n = 5
    matrix_a = phi(cross_sq_dist(c, c))
    ones = jnp.ones((1, n, 1), dtype=c.dtype)
    matrix_b = jnp.concatenate([c, ones], axis=2)
    left = jnp.concatenate([matrix_a, jnp.transpose(matrix_b, (0, 2, 1))], axis=1)
    nb = matrix_b.shape[2]
    right = jnp.concatenate([matrix_b, jnp.zeros((1, nb, nb), dtype=c.dtype)], axis=1)
    lhs = jnp.concatenate([left, right], axis=2)
    rhs = jnp.concatenate([flows, jnp.zeros((1, nb, 2), dtype=c.dtype)], axis=1)
    X = jnp.linalg.solve(lhs, rhs)
    w_c, v_c = X[:, :n, :], X[:, n:, :]

    yg, xg = jnp.meshgrid(jnp.linspace(0.0, H - 1.0, H),
                          jnp.linspace(0.0, W - 1.0, W), indexing='ij')
    grid = jnp.stack([yg, xg], axis=-1).reshape(H * W, 2).astype(jnp.float32)[None]
    pd = phi(cross_sq_dist(grid, c))
    rbf = jnp.einsum('bmn,bnk->bmk', pd, w_c)
    qp = jnp.concatenate([grid, jnp.ones_like(grid[..., :1])], axis=2)
    lin = jnp.einsum('bmd,bdk->bmk', qp, v_c)
    flow = (rbf + lin).reshape(H, W, 2)
    return xg.astype(jnp.float32) - flow[..., 1]


_QTAB = np.asarray(jax.jit(_build_qtab)())
_QTAB_CHUNKED = np.ascontiguousarray(
    _QTAB.reshape(16, ROWS_PER_WORKER, 2, COLS_PER_WORKER)
    .swapaxes(1, 2)).reshape(-1)


def _warp_body(mel_hbm, q_hbm, out_hbm, mel2_v, q_v, out_v):
    wid = lax.axis_index('s') * 2 + lax.axis_index('c')
    i = wid // 2
    j = wid % 2
    row0 = i * ROWS_PER_WORKER
    zvec = jnp.zeros((LANES,), jnp.float32)
    lane = lax.iota(jnp.int32, LANES)

    ct = j * (COLS_PER_WORKER - 128)
    pltpu.sync_copy(
        mel_hbm.at[0, pl.ds(row0, ROWS_PER_WORKER), pl.ds(ct, HALO_W)],
        mel2_v)
    pltpu.sync_copy(q_hbm.at[pl.ds(wid * _CHUNK, _CHUNK)], q_v)

    @plsc.parallel_loop(0, _CHUNK, LANES, unroll=4)
    def _(s):
        r = s // COLS_PER_WORKER
        c = s % COLS_PER_WORKER
        q = q_v[pl.ds(s, LANES)]
        fxi = jnp.clip(q.astype(jnp.int32), 0, W - 2)
        ax = jnp.clip(q - fxi.astype(jnp.float32), 0.0, 1.0)
        rvec = jnp.full((LANES,), r, jnp.int32)
        lin = fxi - ct
        g0 = plsc.load_gather(mel2_v, [rvec, lin])
        g1 = plsc.load_gather(mel2_v, [rvec, lin + 1])
        out_v[r, pl.ds(c, LANES)] = ax * (g1 - g0) + g0

    zs = ze = None
    for lo, hi in _ROW_MASKS:
        s_ = jnp.clip(lo - row0, 0, ROWS_PER_WORKER)
        e_ = jnp.clip(hi - row0, 0, ROWS_PER_WORKER)
        if zs is None:
            zs, ze = s_, e_
        else:
            nonempty = ze > zs
            zs = jnp.where(nonempty, zs, s_)
            ze = jnp.where(nonempty, ze, e_)

    @plsc.parallel_loop(zs * COLS_PER_WORKER, ze * COLS_PER_WORKER, LANES)
    def _(s):
        r = s // COLS_PER_WORKER
        out_v[r, pl.ds(s % COLS_PER_WORKER, LANES)] = zvec

    clo = jnp.where(j == 0, _COL_MASKS[0][0], _COL_MASKS[1][0] - COLS_PER_WORKER)
    for r in range(ROWS_PER_WORKER):
        out_v[r, pl.ds(clo, LANES)] = zvec
        out_v[r, pl.ds(clo + LANES, LANES)] = zvec
        tail = clo + 2 * LANES
        cur = out_v[r, pl.ds(tail, LANES)]
        out_v[r, pl.ds(tail, LANES)] = jnp.where(lane < _T - 2 * LANES, 0.0, cur)

    pltpu.sync_copy(
        out_v,
        out_hbm.at[0, pl.ds(row0, ROWS_PER_WORKER),
                   pl.ds(j * COLS_PER_WORKER, COLS_PER_WORKER)])


@functools.cache
def _warp():
    return pl.kernel(
        _warp_body,
        mesh=plsc.VectorSubcoreMesh(core_axis_name='c', subcore_axis_name='s'),
        compiler_params=pltpu.CompilerParams(
            needs_layout_passes=False,
            skip_device_barrier=True,
            disable_semaphore_checks=True,
        ),
        out_type=jax.ShapeDtypeStruct((1, H, W), jnp.float32),
        scratch_types=[
            pltpu.VMEM((ROWS_PER_WORKER, HALO_W), jnp.float32),
            pltpu.VMEM((_CHUNK,), jnp.float32),
            pltpu.VMEM((ROWS_PER_WORKER, COLS_PER_WORKER), jnp.float32),
        ],
    )


def kernel(mel_spectrogram):
    return _warp()(mel_spectrogram, jnp.asarray(_QTAB_CHUNKED))


# speedup vs baseline: 1.0084x; 1.0084x over previous
import functools

import jax
import jax.numpy as jnp
import numpy as np
from jax import lax
from jax.experimental import pallas as pl
from jax.experimental.pallas import tpu as pltpu
from jax.experimental.pallas import tpu_sc as plsc

H = 128
W = 2048
TIME_WARP_PARA = 40
FREQ_MASK_PARA = 27
TIME_MASK_PARA = 70
FREQ_MASK_NUM = 2
TIME_MASK_NUM = 2

NUM_WORKERS = 32
LANES = 16

ROWS_PER_WORKER = 8
COLS_PER_WORKER = W // 2
_CHUNK = ROWS_PER_WORKER * COLS_PER_WORKER
HALO_W = COLS_PER_WORKER + 128

_F = FREQ_MASK_PARA // 2
_T = TIME_MASK_PARA // 2
_ROW_MASKS = [((i + 1) * H // 4, (i + 1) * H // 4 + _F) for i in range(FREQ_MASK_NUM)]
_COL_MASKS = [((i + 1) * W // 4, (i + 1) * W // 4 + _T) for i in range(TIME_MASK_NUM)]


def _build_qtab():
    eps = 1e-10

    def phi(r):
        r = jnp.maximum(r, eps)
        return 0.5 * r * jnp.log(r)

    def cross_sq_dist(a, b):
        an = jnp.sum(a * a, axis=-1)[:, :, None]
        bn = jnp.sum(b * b, axis=-1)[:, None, :]
        ab = jnp.einsum('bnd,bmd->bnm', a, b)
        return an - 2.0 * ab + bn

    y = float(H // 2)
    pt = float(W // 2)
    dist = float(TIME_WARP_PARA // 2)
    src = jnp.array(
        [[[y, pt], [0.0, 0.0], [0.0, W - 1.0], [H - 1.0, 0.0], [H - 1.0, W - 1.0]]],
        dtype=jnp.float32)
    dst = src.at[0, 0, 1].add(dist)
    flows = dst - src

    c = dst
    n = 5
    matrix_a = phi(cross_sq_dist(c, c))
    ones = jnp.ones((1, n, 1), dtype=c.dtype)
    matrix_b = jnp.concatenate([c, ones], axis=2)
    left = jnp.concatenate([matrix_a, jnp.transpose(matrix_b, (0, 2, 1))], axis=1)
    nb = matrix_b.shape[2]
    right = jnp.concatenate([matrix_b, jnp.zeros((1, nb, nb), dtype=c.dtype)], axis=1)
    lhs = jnp.concatenate([left, right], axis=2)
    rhs = jnp.concatenate([flows, jnp.zeros((1, nb, 2), dtype=c.dtype)], axis=1)
    X = jnp.linalg.solve(lhs, rhs)
    w_c, v_c = X[:, :n, :], X[:, n:, :]

    yg, xg = jnp.meshgrid(jnp.linspace(0.0, H - 1.0, H),
                          jnp.linspace(0.0, W - 1.0, W), indexing='ij')
    grid = jnp.stack([yg, xg], axis=-1).reshape(H * W, 2).astype(jnp.float32)[None]
    pd = phi(cross_sq_dist(grid, c))
    rbf = jnp.einsum('bmn,bnk->bmk', pd, w_c)
    qp = jnp.concatenate([grid, jnp.ones_like(grid[..., :1])], axis=2)
    lin = jnp.einsum('bmd,bdk->bmk', qp, v_c)
    flow = (rbf + lin).reshape(H, W, 2)
    return xg.astype(jnp.float32) - flow[..., 1]


_QTAB = np.asarray(jax.jit(_build_qtab)())
_QTAB_CHUNKED = np.ascontiguousarray(
    _QTAB.reshape(16, ROWS_PER_WORKER, 2, COLS_PER_WORKER)
    .swapaxes(1, 2)).reshape(-1)


def _warp_body(mel_hbm, q_hbm, out_hbm, mel_v, q_v, out_v, sem):
    wid = lax.axis_index('s') * 2 + lax.axis_index('c')
    i = wid // 2
    j = wid % 2
    row0 = i * ROWS_PER_WORKER
    zvec = jnp.zeros((LANES,), jnp.float32)
    lane = lax.iota(jnp.int32, LANES)

    ct = j * (COLS_PER_WORKER - 128)
    copies = [
        pltpu.async_copy(
            mel_hbm.at[0, row0 + r, pl.ds(ct, HALO_W)],
            mel_v.at[pl.ds(r * HALO_W, HALO_W)], sem)
        for r in range(ROWS_PER_WORKER)
    ]
    copies.append(
        pltpu.async_copy(q_hbm.at[pl.ds(wid * _CHUNK, _CHUNK)], q_v, sem))
    for cp in copies:
        cp.wait()

    for r in range(ROWS_PER_WORKER):
        rbase = r * HALO_W - ct

        @plsc.parallel_loop(0, COLS_PER_WORKER, LANES, unroll=4)
        def _(c, r=r, rbase=rbase):
            q = q_v[pl.ds(r * COLS_PER_WORKER + c, LANES)]
            fxi = jnp.clip(q.astype(jnp.int32), 0, W - 2)
            ax = jnp.clip(q - fxi.astype(jnp.float32), 0.0, 1.0)
            lin = fxi + rbase
            g0 = plsc.load_gather(mel_v, [lin])
            g1 = plsc.load_gather(mel_v, [lin + 1])
            out_v[r, pl.ds(c, LANES)] = ax * (g1 - g0) + g0

    zs = ze = None
    for lo, hi in _ROW_MASKS:
        s_ = jnp.clip(lo - row0, 0, ROWS_PER_WORKER)
        e_ = jnp.clip(hi - row0, 0, ROWS_PER_WORKER)
        if zs is None:
            zs, ze = s_, e_
        else:
            nonempty = ze > zs
            zs = jnp.where(nonempty, zs, s_)
            ze = jnp.where(nonempty, ze, e_)

    @plsc.parallel_loop(zs * COLS_PER_WORKER, ze * COLS_PER_WORKER, LANES)
    def _(s):
        r = s // COLS_PER_WORKER
        out_v[r, pl.ds(s % COLS_PER_WORKER, LANES)] = zvec

    clo = jnp.where(j == 0, _COL_MASKS[0][0], _COL_MASKS[1][0] - COLS_PER_WORKER)
    for r in range(ROWS_PER_WORKER):
        out_v[r, pl.ds(clo, LANES)] = zvec
        out_v[r, pl.ds(clo + LANES, LANES)] = zvec
        tail = clo + 2 * LANES
        cur = out_v[r, pl.ds(tail, LANES)]
        out_v[r, pl.ds(tail, LANES)] = jnp.where(lane < _T - 2 * LANES, 0.0, cur)

    pltpu.sync_copy(
        out_v,
        out_hbm.at[0, pl.ds(row0, ROWS_PER_WORKER),
                   pl.ds(j * COLS_PER_WORKER, COLS_PER_WORKER)])


@functools.cache
def _warp():
    return pl.kernel(
        _warp_body,
        mesh=plsc.VectorSubcoreMesh(core_axis_name='c', subcore_axis_name='s'),
        compiler_params=pltpu.CompilerParams(needs_layout_passes=False),
        out_type=jax.ShapeDtypeStruct((1, H, W), jnp.float32),
        scratch_types=[
            pltpu.VMEM((ROWS_PER_WORKER * HALO_W,), jnp.float32),
            pltpu.VMEM((_CHUNK,), jnp.float32),
            pltpu.VMEM((ROWS_PER_WORKER, COLS_PER_WORKER), jnp.float32),
            pltpu.SemaphoreType.DMA,
        ],
    )


def kernel(mel_spectrogram):
    return _warp()(mel_spectrogram, jnp.asarray(_QTAB_CHUNKED))


# speedup vs baseline: 1.0212x; 1.0127x over previous
import functools

import jax
import jax.numpy as jnp
import numpy as np
from jax import lax
from jax.experimental import pallas as pl
from jax.experimental.pallas import tpu as pltpu
from jax.experimental.pallas import tpu_sc as plsc

H = 128
W = 2048
TIME_WARP_PARA = 40
FREQ_MASK_PARA = 27
TIME_MASK_PARA = 70
FREQ_MASK_NUM = 2
TIME_MASK_NUM = 2

NUM_WORKERS = 32
LANES = 16

ROWS_PER_WORKER = 8
COLS_PER_WORKER = W // 2
_CHUNK = ROWS_PER_WORKER * COLS_PER_WORKER
HALO_W = COLS_PER_WORKER + 128

_F = FREQ_MASK_PARA // 2
_T = TIME_MASK_PARA // 2
_ROW_MASKS = [((i + 1) * H // 4, (i + 1) * H // 4 + _F) for i in range(FREQ_MASK_NUM)]
_COL_MASKS = [((i + 1) * W // 4, (i + 1) * W // 4 + _T) for i in range(TIME_MASK_NUM)]


def _build_qtab():
    eps = 1e-10

    def phi(r):
        r = jnp.maximum(r, eps)
        return 0.5 * r * jnp.log(r)

    def cross_sq_dist(a, b):
        an = jnp.sum(a * a, axis=-1)[:, :, None]
        bn = jnp.sum(b * b, axis=-1)[:, None, :]
        ab = jnp.einsum('bnd,bmd->bnm', a, b)
        return an - 2.0 * ab + bn

    y = float(H // 2)
    pt = float(W // 2)
    dist = float(TIME_WARP_PARA // 2)
    src = jnp.array(
        [[[y, pt], [0.0, 0.0], [0.0, W - 1.0], [H - 1.0, 0.0], [H - 1.0, W - 1.0]]],
        dtype=jnp.float32)
    dst = src.at[0, 0, 1].add(dist)
    flows = dst - src

    c = dst
    n = 5
    matrix_a = phi(cross_sq_dist(c, c))
    ones = jnp.ones((1, n, 1), dtype=c.dtype)
    matrix_b = jnp.concatenate([c, ones], axis=2)
    left = jnp.concatenate([matrix_a, jnp.transpose(matrix_b, (0, 2, 1))], axis=1)
    nb = matrix_b.shape[2]
    right = jnp.concatenate([matrix_b, jnp.zeros((1, nb, nb), dtype=c.dtype)], axis=1)
    lhs = jnp.concatenate([left, right], axis=2)
    rhs = jnp.concatenate([flows, jnp.zeros((1, nb, 2), dtype=c.dtype)], axis=1)
    X = jnp.linalg.solve(lhs, rhs)
    w_c, v_c = X[:, :n, :], X[:, n:, :]

    yg, xg = jnp.meshgrid(jnp.linspace(0.0, H - 1.0, H),
                          jnp.linspace(0.0, W - 1.0, W), indexing='ij')
    grid = jnp.stack([yg, xg], axis=-1).reshape(H * W, 2).astype(jnp.float32)[None]
    pd = phi(cross_sq_dist(grid, c))
    rbf = jnp.einsum('bmn,bnk->bmk', pd, w_c)
    qp = jnp.concatenate([grid, jnp.ones_like(grid[..., :1])], axis=2)
    lin = jnp.einsum('bmd,bdk->bmk', qp, v_c)
    flow = (rbf + lin).reshape(H, W, 2)
    return xg.astype(jnp.float32) - flow[..., 1]


_QTAB = np.asarray(jax.jit(_build_qtab)())


def _build_lin_ax():
    q = _QTAB
    fx = np.clip(np.trunc(q).astype(np.int64), 0, W - 2)
    ax = np.clip(q - fx.astype(np.float32), 0.0, 1.0).astype(np.float32)
    r_local = (np.arange(H) % ROWS_PER_WORKER)[:, None]
    ct = (np.arange(W) // COLS_PER_WORKER) * (COLS_PER_WORKER - 128)
    lin = (r_local * HALO_W + fx - ct[None, :]).astype(np.int32)

    def chunked(t):
        return np.ascontiguousarray(
            t.reshape(16, ROWS_PER_WORKER, 2, COLS_PER_WORKER)
            .swapaxes(1, 2)).reshape(-1)

    return chunked(lin), chunked(ax)


_LIN_CHUNKED, _AX_CHUNKED = _build_lin_ax()


def _warp_body(mel_hbm, lin_hbm, ax_hbm, out_hbm, mel_v, lin_v, ax_v, out_v, sem):
    wid = lax.axis_index('s') * 2 + lax.axis_index('c')
    i = wid // 2
    j = wid % 2
    row0 = i * ROWS_PER_WORKER
    zvec = jnp.zeros((LANES,), jnp.float32)
    lane = lax.iota(jnp.int32, LANES)

    ct = j * (COLS_PER_WORKER - 128)
    copies = [
        pltpu.async_copy(
            mel_hbm.at[0, row0 + r, pl.ds(ct, HALO_W)],
            mel_v.at[pl.ds(r * HALO_W, HALO_W)], sem)
        for r in range(ROWS_PER_WORKER)
    ]
    copies.append(
        pltpu.async_copy(lin_hbm.at[pl.ds(wid * _CHUNK, _CHUNK)], lin_v, sem))
    copies.append(
        pltpu.async_copy(ax_hbm.at[pl.ds(wid * _CHUNK, _CHUNK)], ax_v, sem))
    for cp in copies:
        cp.wait()

    for r in range(ROWS_PER_WORKER):
        @plsc.parallel_loop(0, COLS_PER_WORKER, LANES, unroll=8)
        def _(c, r=r):
            s = r * COLS_PER_WORKER + c
            lin = lin_v[pl.ds(s, LANES)]
            ax = ax_v[pl.ds(s, LANES)]
            g0 = plsc.load_gather(mel_v, [lin])
            g1 = plsc.load_gather(mel_v, [lin + 1])
            out_v[r, pl.ds(c, LANES)] = ax * (g1 - g0) + g0

    zs = ze = None
    for lo, hi in _ROW_MASKS:
        s_ = jnp.clip(lo - row0, 0, ROWS_PER_WORKER)
        e_ = jnp.clip(hi - row0, 0, ROWS_PER_WORKER)
        if zs is None:
            zs, ze = s_, e_
        else:
            nonempty = ze > zs
            zs = jnp.where(nonempty, zs, s_)
            ze = jnp.where(nonempty, ze, e_)

    @plsc.parallel_loop(zs * COLS_PER_WORKER, ze * COLS_PER_WORKER, LANES)
    def _(s):
        r = s // COLS_PER_WORKER
        out_v[r, pl.ds(s % COLS_PER_WORKER, LANES)] = zvec

    clo = jnp.where(j == 0, _COL_MASKS[0][0], _COL_MASKS[1][0] - COLS_PER_WORKER)
    for r in range(ROWS_PER_WORKER):
        out_v[r, pl.ds(clo, LANES)] = zvec
        out_v[r, pl.ds(clo + LANES, LANES)] = zvec
        tail = clo + 2 * LANES
        cur = out_v[r, pl.ds(tail, LANES)]
        out_v[r, pl.ds(tail, LANES)] = jnp.where(lane < _T - 2 * LANES, 0.0, cur)

    pltpu.sync_copy(
        out_v,
        out_hbm.at[0, pl.ds(row0, ROWS_PER_WORKER),
                   pl.ds(j * COLS_PER_WORKER, COLS_PER_WORKER)])


@functools.cache
def _warp():
    return pl.kernel(
        _warp_body,
        mesh=plsc.VectorSubcoreMesh(core_axis_name='c', subcore_axis_name='s'),
        compiler_params=pltpu.CompilerParams(needs_layout_passes=False),
        out_type=jax.ShapeDtypeStruct((1, H, W), jnp.float32),
        scratch_types=[
            pltpu.VMEM((ROWS_PER_WORKER * HALO_W,), jnp.float32),
            pltpu.VMEM((_CHUNK,), jnp.int32),
            pltpu.VMEM((_CHUNK,), jnp.float32),
            pltpu.VMEM((ROWS_PER_WORKER, COLS_PER_WORKER), jnp.float32),
            pltpu.SemaphoreType.DMA,
        ],
    )


def kernel(mel_spectrogram):
    return _warp()(mel_spectrogram, jnp.asarray(_LIN_CHUNKED),
                   jnp.asarray(_AX_CHUNKED))
